# baseline (speedup 1.0000x reference)
import functools

import jax
import jax.numpy as jnp
from jax import lax
from jax.experimental import pallas as pl
from jax.experimental.pallas import tpu as pltpu
from jax.experimental.pallas import tpu_sc as plsc

N_NODES = 10000
N_EDGES = 160000
D = 256
DW = D // 2

NW = 32
EPW = N_EDGES // NW
C = 80
NCHUNK = EPW // C
TAIL = EPW - NCHUNK * C


def _sc_body(new_hbm, raw_hbm, src_hbm, dst_hbm, out_hbm,
             src_v, dst_v, u0, u1, v0, v1, ut, vt, scores_v,
             sem0, sem1, semt):
    wid = lax.axis_index("s") * 2 + lax.axis_index("c")
    base = wid * EPW

    pltpu.sync_copy(src_hbm.at[pl.ds(base, EPW)], src_v)
    pltpu.sync_copy(dst_hbm.at[pl.ds(base, EPW)], dst_v)

    ubufs = (u0, u1)
    vbufs = (v0, v1)
    sems = (sem0, sem1)

    def fire(j, b):
        pltpu.async_copy(new_hbm.at[src_v.at[pl.ds(j * C, C)]], ubufs[b],
                         sems[b])
        pltpu.async_copy(raw_hbm.at[dst_v.at[pl.ds(j * C, C)]], vbufs[b],
                         sems[b])

    def wait(j, b):
        pltpu.make_async_copy(new_hbm.at[src_v.at[pl.ds(j * C, C)]],
                              ubufs[b], sems[b]).wait()
        pltpu.make_async_copy(raw_hbm.at[dst_v.at[pl.ds(j * C, C)]],
                              vbufs[b], sems[b]).wait()

    fire(0, 0)
    pltpu.async_copy(new_hbm.at[src_v.at[pl.ds(NCHUNK * C, TAIL)]], ut, semt)
    pltpu.async_copy(raw_hbm.at[dst_v.at[pl.ds(NCHUNK * C, TAIL)]], vt, semt)

    iota = lax.iota(jnp.int32, 16)

    def dot_group(ub, vb, idx_e):
        acc0 = jnp.zeros((16,), jnp.float32)

        def feat_body(w, carry):
            acc_lo, acc_hi = carry
            idx_w = (iota + w) % 16 + (w - w % 16)
            uw = plsc.load_gather(ub, [idx_e, idx_w])
            vw = plsc.load_gather(vb, [idx_e, idx_w])
            p = plsc.bitcast(uw, jnp.bfloat16) * plsc.bitcast(vw, jnp.bfloat16)
            lo, hi = plsc.unpack(p, format=plsc.PackFormat.INTERLEAVED)
            return (acc_lo + lo, acc_hi + hi)

        acc_lo, acc_hi = pl.loop(0, DW, init_carry=(acc0, acc0),
                                 unroll=8)(feat_body)
        return acc_lo + acc_hi

    def compute(j, b):
        for g in range(C // 16):
            idx_e = iota + (g * 16)
            acc = dot_group(ubufs[b], vbufs[b], idx_e)
            scores_v[pl.ds(j * C + g * 16, 16)] = acc

    def outer(j0):
        for b in range(2):
            j = j0 + b

            @pl.when(j < NCHUNK - 1)
            def _():
                fire(j + 1, 1 - b)

            wait(j, b)
            compute(j, b)

    pl.loop(0, NCHUNK, step=2)(outer)

    pltpu.make_async_copy(new_hbm.at[src_v.at[pl.ds(NCHUNK * C, TAIL)]],
                          ut, semt).wait()
    pltpu.make_async_copy(raw_hbm.at[dst_v.at[pl.ds(NCHUNK * C, TAIL)]],
                          vt, semt).wait()
    for g in range(3):
        idx_e = jnp.minimum(iota + (g * 16), TAIL - 1)
        acc = dot_group(ut, vt, idx_e)
        scores_v[pl.ds(NCHUNK * C + g * 16, 16)] = acc

    pltpu.sync_copy(scores_v.at[pl.ds(0, EPW)], out_hbm.at[pl.ds(base, EPW)])


@jax.jit
def _run(new_ft, raw_ft, src, dst):
    mesh = plsc.VectorSubcoreMesh(core_axis_name="c", subcore_axis_name="s")
    kfn = pl.kernel(
        _sc_body,
        out_type=jax.ShapeDtypeStruct((N_EDGES,), jnp.float32),
        mesh=mesh,
        compiler_params=pltpu.CompilerParams(use_tc_tiling_on_sc=False,
                                             needs_layout_passes=False),
        scratch_types=[
            pltpu.VMEM((EPW,), jnp.int32),
            pltpu.VMEM((EPW,), jnp.int32),
            pltpu.VMEM((C, DW), jnp.int32),
            pltpu.VMEM((C, DW), jnp.int32),
            pltpu.VMEM((C, DW), jnp.int32),
            pltpu.VMEM((C, DW), jnp.int32),
            pltpu.VMEM((TAIL, DW), jnp.int32),
            pltpu.VMEM((TAIL, DW), jnp.int32),
            pltpu.VMEM((EPW + 8,), jnp.float32),
            pltpu.SemaphoreType.DMA,
            pltpu.SemaphoreType.DMA,
            pltpu.SemaphoreType.DMA,
        ],
    )
    return kfn(new_ft, raw_ft, src, dst)


def _pack_bf16(x):
    xb = x.astype(jnp.bfloat16).reshape(N_NODES, DW, 2)
    return jax.lax.bitcast_convert_type(xb, jnp.int32)


def kernel(new_ft, raw_ft, edge_index):
    ei = edge_index.astype(jnp.int32)
    scores = _run(_pack_bf16(new_ft), _pack_bf16(raw_ft), ei[0], ei[1])
    return scores.reshape(N_EDGES, 1)


# speedup vs baseline: 2.4708x; 2.4708x over previous
import functools

import jax
import jax.numpy as jnp
from jax import lax
from jax.experimental import pallas as pl
from jax.experimental.pallas import tpu as pltpu
from jax.experimental.pallas import tpu_sc as plsc

N_NODES = 10000
N_EDGES = 160000
D = 256
DW = D // 2

NW = 32
EPW = N_EDGES // NW
C = 80
NCHUNK = EPW // C
TAIL = EPW - NCHUNK * C


def _sc_body(new_hbm, raw_hbm, src_hbm, dst_hbm, out_hbm,
             src_v, dst_v, u0, u1, v0, v1, ut, vt, scores_v,
             sem0, sem1, semt):
    wid = lax.axis_index("s") * 2 + lax.axis_index("c")
    base = wid * EPW

    pltpu.sync_copy(src_hbm.at[pl.ds(base, EPW)], src_v)
    pltpu.sync_copy(dst_hbm.at[pl.ds(base, EPW)], dst_v)

    ubufs = (u0, u1)
    vbufs = (v0, v1)
    sems = (sem0, sem1)

    def fire(j, b):
        pltpu.async_copy(new_hbm.at[src_v.at[pl.ds(j * C, C)]], ubufs[b],
                         sems[b])
        pltpu.async_copy(raw_hbm.at[dst_v.at[pl.ds(j * C, C)]], vbufs[b],
                         sems[b])

    def wait(j, b):
        pltpu.make_async_copy(new_hbm.at[src_v.at[pl.ds(j * C, C)]],
                              ubufs[b], sems[b]).wait()
        pltpu.make_async_copy(raw_hbm.at[dst_v.at[pl.ds(j * C, C)]],
                              vbufs[b], sems[b]).wait()

    fire(0, 0)
    pltpu.async_copy(new_hbm.at[src_v.at[pl.ds(NCHUNK * C, TAIL)]], ut, semt)
    pltpu.async_copy(raw_hbm.at[dst_v.at[pl.ds(NCHUNK * C, TAIL)]], vt, semt)

    iota = lax.iota(jnp.int32, 16)

    def dot_group(ub, vb, idx_e):
        acc0 = jnp.zeros((16,), jnp.float32)

        def feat_body(w, carry):
            acc_lo, acc_hi = carry
            idx_w = (iota + w) % 16 + (w - w % 16)
            uw = plsc.load_gather(ub, [idx_e, idx_w])
            vw = plsc.load_gather(vb, [idx_e, idx_w])
            p = plsc.bitcast(uw, jnp.bfloat16) * plsc.bitcast(vw, jnp.bfloat16)
            lo, hi = plsc.unpack(p, format=plsc.PackFormat.INTERLEAVED)
            return (acc_lo + lo, acc_hi + hi)

        acc_lo, acc_hi = pl.loop(0, DW, init_carry=(acc0, acc0),
                                 unroll=8)(feat_body)
        return acc_lo + acc_hi

    def compute(j, b):
        for g in range(C // 16):
            idx_e = iota + (g * 16)
            acc = dot_group(ubufs[b], vbufs[b], idx_e)
            scores_v[pl.ds(j * C + g * 16, 16)] = acc

    def outer(j0):
        for b in range(2):
            j = j0 + b

            @pl.when(j < NCHUNK - 1)
            def _():
                fire(j + 1, 1 - b)

            wait(j, b)
            compute(j, b)

    pl.loop(0, NCHUNK, step=2)(outer)

    pltpu.make_async_copy(new_hbm.at[src_v.at[pl.ds(NCHUNK * C, TAIL)]],
                          ut, semt).wait()
    pltpu.make_async_copy(raw_hbm.at[dst_v.at[pl.ds(NCHUNK * C, TAIL)]],
                          vt, semt).wait()
    for g in range(3):
        idx_e = jnp.minimum(iota + (g * 16), TAIL - 1)
        acc = dot_group(ut, vt, idx_e)
        scores_v[pl.ds(NCHUNK * C + g * 16, 16)] = acc

    pltpu.sync_copy(scores_v.at[pl.ds(0, EPW)], out_hbm.at[pl.ds(base, EPW)])


@jax.jit
def _run(new_ft, raw_ft, src, dst):
    mesh = plsc.VectorSubcoreMesh(core_axis_name="c", subcore_axis_name="s")
    kfn = pl.kernel(
        _sc_body,
        out_type=jax.ShapeDtypeStruct((N_EDGES,), jnp.float32),
        mesh=mesh,
        compiler_params=pltpu.CompilerParams(use_tc_tiling_on_sc=False,
                                             needs_layout_passes=False),
        scratch_types=[
            pltpu.VMEM((EPW,), jnp.int32),
            pltpu.VMEM((EPW,), jnp.int32),
            pltpu.VMEM((C, DW), jnp.int32),
            pltpu.VMEM((C, DW), jnp.int32),
            pltpu.VMEM((C, DW), jnp.int32),
            pltpu.VMEM((C, DW), jnp.int32),
            pltpu.VMEM((TAIL, DW), jnp.int32),
            pltpu.VMEM((TAIL, DW), jnp.int32),
            pltpu.VMEM((EPW + 8,), jnp.float32),
            pltpu.SemaphoreType.DMA,
            pltpu.SemaphoreType.DMA,
            pltpu.SemaphoreType.DMA,
        ],
    )
    return kfn(new_ft, raw_ft, src, dst)


def _pack_bf16(x):
    u = jax.lax.bitcast_convert_type(x, jnp.uint32)
    b = (u + 0x7FFF + ((u >> 16) & 1)) >> 16
    packed = b[:, :DW] | (b[:, DW:] << 16)
    return jax.lax.bitcast_convert_type(packed, jnp.int32)


def kernel(new_ft, raw_ft, edge_index):
    ei = edge_index.astype(jnp.int32)
    scores = _run(_pack_bf16(new_ft), _pack_bf16(raw_ft), ei[0], ei[1])
    return scores.reshape(N_EDGES, 1)


# speedup vs baseline: 2.4832x; 1.0050x over previous
import functools

import jax
import jax.numpy as jnp
from jax import lax
from jax.experimental import pallas as pl
from jax.experimental.pallas import tpu as pltpu
from jax.experimental.pallas import tpu_sc as plsc

N_NODES = 10000
N_EDGES = 160000
D = 256
DW = D // 2

NW = 32
EPW = N_EDGES // NW
C = 80
NCHUNK = EPW // C
TAIL = EPW - NCHUNK * C


def _sc_body(new_hbm, raw_hbm, src_hbm, dst_hbm, out_hbm,
             src_v, dst_v, u0, u1, v0, v1, ut, vt, scores_v,
             sem0, sem1, semt):
    wid = lax.axis_index("s") * 2 + lax.axis_index("c")
    base = wid * EPW

    pltpu.sync_copy(src_hbm.at[pl.ds(base, EPW)], src_v)
    pltpu.sync_copy(dst_hbm.at[pl.ds(base, EPW)], dst_v)

    ubufs = (u0, u1)
    vbufs = (v0, v1)
    sems = (sem0, sem1)

    def fire(j, b):
        pltpu.async_copy(new_hbm.at[src_v.at[pl.ds(j * C, C)]], ubufs[b],
                         sems[b])
        pltpu.async_copy(raw_hbm.at[dst_v.at[pl.ds(j * C, C)]], vbufs[b],
                         sems[b])

    def wait(j, b):
        pltpu.make_async_copy(new_hbm.at[src_v.at[pl.ds(j * C, C)]],
                              ubufs[b], sems[b]).wait()
        pltpu.make_async_copy(raw_hbm.at[dst_v.at[pl.ds(j * C, C)]],
                              vbufs[b], sems[b]).wait()

    fire(0, 0)
    pltpu.async_copy(new_hbm.at[src_v.at[pl.ds(NCHUNK * C, TAIL)]], ut, semt)
    pltpu.async_copy(raw_hbm.at[dst_v.at[pl.ds(NCHUNK * C, TAIL)]], vt, semt)

    iota = lax.iota(jnp.int32, 16)

    def dot_group(ub, vb, idx_e):
        acc0 = jnp.zeros((16,), jnp.float32)

        def feat_body(w, carry):
            acc_lo, acc_hi = carry
            idx_w = (iota + w) % 16 + (w - w % 16)
            uw = plsc.load_gather(ub, [idx_e, idx_w])
            vw = plsc.load_gather(vb, [idx_e, idx_w])
            p = plsc.bitcast(uw, jnp.bfloat16) * plsc.bitcast(vw, jnp.bfloat16)
            lo, hi = plsc.unpack(p, format=plsc.PackFormat.INTERLEAVED)
            return (acc_lo + lo, acc_hi + hi)

        acc_lo, acc_hi = pl.loop(0, DW, init_carry=(acc0, acc0),
                                 unroll=8)(feat_body)
        return acc_lo + acc_hi

    def compute(j, b):
        for g in range(C // 16):
            idx_e = iota + (g * 16)
            acc = dot_group(ubufs[b], vbufs[b], idx_e)
            scores_v[pl.ds(j * C + g * 16, 16)] = acc

    def outer(j0):
        for b in range(2):
            j = j0 + b

            @pl.when(j < NCHUNK - 1)
            def _():
                fire(j + 1, 1 - b)

            wait(j, b)
            compute(j, b)

    pl.loop(0, NCHUNK, step=2)(outer)

    pltpu.make_async_copy(new_hbm.at[src_v.at[pl.ds(NCHUNK * C, TAIL)]],
                          ut, semt).wait()
    pltpu.make_async_copy(raw_hbm.at[dst_v.at[pl.ds(NCHUNK * C, TAIL)]],
                          vt, semt).wait()
    for g in range(3):
        idx_e = jnp.minimum(iota + (g * 16), TAIL - 1)
        acc = dot_group(ut, vt, idx_e)
        scores_v[pl.ds(NCHUNK * C + g * 16, 16)] = acc

    pltpu.sync_copy(scores_v.at[pl.ds(0, EPW)], out_hbm.at[pl.ds(base, EPW)])


@jax.jit
def _run(new_ft, raw_ft, src, dst):
    mesh = plsc.VectorSubcoreMesh(core_axis_name="c", subcore_axis_name="s")
    kfn = pl.kernel(
        _sc_body,
        out_type=jax.ShapeDtypeStruct((N_EDGES,), jnp.float32),
        mesh=mesh,
        compiler_params=pltpu.CompilerParams(use_tc_tiling_on_sc=False,
                                             needs_layout_passes=False),
        scratch_types=[
            pltpu.VMEM((EPW,), jnp.int32),
            pltpu.VMEM((EPW,), jnp.int32),
            pltpu.VMEM((C, DW), jnp.int32),
            pltpu.VMEM((C, DW), jnp.int32),
            pltpu.VMEM((C, DW), jnp.int32),
            pltpu.VMEM((C, DW), jnp.int32),
            pltpu.VMEM((TAIL, DW), jnp.int32),
            pltpu.VMEM((TAIL, DW), jnp.int32),
            pltpu.VMEM((EPW + 8,), jnp.float32),
            pltpu.SemaphoreType.DMA,
            pltpu.SemaphoreType.DMA,
            pltpu.SemaphoreType.DMA,
        ],
    )
    return kfn(new_ft, raw_ft, src, dst)


def _pack_bf16(x):
    u = jax.lax.bitcast_convert_type(x, jnp.uint32)
    b = (u + 0x8000) >> 16
    packed = b[:, :DW] | (b[:, DW:] << 16)
    return jax.lax.bitcast_convert_type(packed, jnp.int32)


def kernel(new_ft, raw_ft, edge_index):
    ei = edge_index.astype(jnp.int32)
    scores = _run(_pack_bf16(new_ft), _pack_bf16(raw_ft), ei[0], ei[1])
    return scores.reshape(N_EDGES, 1)


# speedup vs baseline: 2.4901x; 1.0028x over previous
import functools

import jax
import jax.numpy as jnp
from jax import lax
from jax.experimental import pallas as pl
from jax.experimental.pallas import tpu as pltpu
from jax.experimental.pallas import tpu_sc as plsc

N_NODES = 10000
N_EDGES = 160000
D = 256
DW = D // 2

NW = 32
EPW = N_EDGES // NW
C = 80
NCHUNK = EPW // C
TAIL = EPW - NCHUNK * C


def _sc_body(new_hbm, raw_hbm, src_hbm, dst_hbm, out_hbm,
             src_v, dst_v, u0, u1, v0, v1, ut, vt, scores_v,
             sem0, sem1, semt):
    wid = lax.axis_index("s") * 2 + lax.axis_index("c")
    base = wid * EPW

    pltpu.sync_copy(src_hbm.at[pl.ds(base, EPW)], src_v)
    pltpu.sync_copy(dst_hbm.at[pl.ds(base, EPW)], dst_v)

    ubufs = (u0, u1)
    vbufs = (v0, v1)
    sems = (sem0, sem1)

    def fire(j, b):
        pltpu.async_copy(new_hbm.at[src_v.at[pl.ds(j * C, C)]], ubufs[b],
                         sems[b])
        pltpu.async_copy(raw_hbm.at[dst_v.at[pl.ds(j * C, C)]], vbufs[b],
                         sems[b])

    def wait(j, b):
        pltpu.make_async_copy(new_hbm.at[src_v.at[pl.ds(j * C, C)]],
                              ubufs[b], sems[b]).wait()
        pltpu.make_async_copy(raw_hbm.at[dst_v.at[pl.ds(j * C, C)]],
                              vbufs[b], sems[b]).wait()

    fire(0, 0)
    pltpu.async_copy(new_hbm.at[src_v.at[pl.ds(NCHUNK * C, TAIL)]], ut, semt)
    pltpu.async_copy(raw_hbm.at[dst_v.at[pl.ds(NCHUNK * C, TAIL)]], vt, semt)

    iota = lax.iota(jnp.int32, 16)

    def dot_group(ub, vb, idx_e):
        acc0 = jnp.zeros((16,), jnp.float32)

        def feat_body(w, carry):
            acc_lo, acc_hi = carry
            idx_w = (iota + w) % 16 + (w - w % 16)
            uw = plsc.load_gather(ub, [idx_e, idx_w])
            vw = plsc.load_gather(vb, [idx_e, idx_w])
            p = plsc.bitcast(uw, jnp.bfloat16) * plsc.bitcast(vw, jnp.bfloat16)
            lo, hi = plsc.unpack(p, format=plsc.PackFormat.INTERLEAVED)
            return (acc_lo + lo, acc_hi + hi)

        acc_lo, acc_hi = pl.loop(0, DW, init_carry=(acc0, acc0),
                                 unroll=8)(feat_body)
        return acc_lo + acc_hi

    def compute(j, b):
        for g in range(C // 16):
            idx_e = iota + (g * 16)
            acc = dot_group(ubufs[b], vbufs[b], idx_e)
            scores_v[pl.ds(j * C + g * 16, 16)] = acc

    def outer(j0):
        for b in range(2):
            j = j0 + b

            @pl.when(j < NCHUNK - 1)
            def _():
                fire(j + 1, 1 - b)

            wait(j, b)
            compute(j, b)

    pl.loop(0, NCHUNK, step=2)(outer)

    pltpu.make_async_copy(new_hbm.at[src_v.at[pl.ds(NCHUNK * C, TAIL)]],
                          ut, semt).wait()
    pltpu.make_async_copy(raw_hbm.at[dst_v.at[pl.ds(NCHUNK * C, TAIL)]],
                          vt, semt).wait()
    for g in range(3):
        idx_e = jnp.minimum(iota + (g * 16), TAIL - 1)
        acc = dot_group(ut, vt, idx_e)
        scores_v[pl.ds(NCHUNK * C + g * 16, 16)] = acc

    pltpu.sync_copy(scores_v.at[pl.ds(0, EPW)], out_hbm.at[pl.ds(base, EPW)])


@jax.jit
def _run(new_ft, raw_ft, src, dst):
    mesh = plsc.VectorSubcoreMesh(core_axis_name="c", subcore_axis_name="s")
    kfn = pl.kernel(
        _sc_body,
        out_type=jax.ShapeDtypeStruct((N_EDGES,), jnp.float32),
        mesh=mesh,
        compiler_params=pltpu.CompilerParams(use_tc_tiling_on_sc=False,
                                             needs_layout_passes=False),
        scratch_types=[
            pltpu.VMEM((EPW,), jnp.int32),
            pltpu.VMEM((EPW,), jnp.int32),
            pltpu.VMEM((C, DW), jnp.int32),
            pltpu.VMEM((C, DW), jnp.int32),
            pltpu.VMEM((C, DW), jnp.int32),
            pltpu.VMEM((C, DW), jnp.int32),
            pltpu.VMEM((TAIL, DW), jnp.int32),
            pltpu.VMEM((TAIL, DW), jnp.int32),
            pltpu.VMEM((EPW + 8,), jnp.float32),
            pltpu.SemaphoreType.DMA,
            pltpu.SemaphoreType.DMA,
            pltpu.SemaphoreType.DMA,
        ],
    )
    return kfn(new_ft, raw_ft, src, dst)


def _pack_bf16(x):
    u = jax.lax.bitcast_convert_type(x, jnp.uint32)
    packed = u[:, :DW]
    return jax.lax.bitcast_convert_type(packed, jnp.int32)


def kernel(new_ft, raw_ft, edge_index):
    ei = edge_index.astype(jnp.int32)
    scores = _run(_pack_bf16(new_ft), _pack_bf16(raw_ft), ei[0], ei[1])
    return scores.reshape(N_EDGES, 1)


# speedup vs baseline: 2.5805x; 1.0363x over previous
import functools

import jax
import jax.numpy as jnp
from jax import lax
from jax.experimental import pallas as pl
from jax.experimental.pallas import tpu as pltpu
from jax.experimental.pallas import tpu_sc as plsc

N_NODES = 10000
N_EDGES = 160000
D = 256
DW = D // 2

NW = 32
EPW = N_EDGES // NW
C = 80
NCHUNK = EPW // C
TAIL = EPW - NCHUNK * C


def _sc_body(new_hbm, raw_hbm, ei_hbm, out_hbm,
             src_v, dst_v, u0, u1, v0, v1, ut, vt, scores_v,
             sem0, sem1, semt):
    wid = lax.axis_index("s") * 2 + lax.axis_index("c")
    base = wid * EPW

    pltpu.sync_copy(ei_hbm.at[0, pl.ds(base, EPW)], src_v)
    pltpu.sync_copy(ei_hbm.at[1, pl.ds(base, EPW)], dst_v)

    ubufs = (u0, u1)
    vbufs = (v0, v1)
    sems = (sem0, sem1)

    def fire(j, b):
        pltpu.async_copy(new_hbm.at[src_v.at[pl.ds(j * C, C)]], ubufs[b],
                         sems[b])
        pltpu.async_copy(raw_hbm.at[dst_v.at[pl.ds(j * C, C)]], vbufs[b],
                         sems[b])

    def wait(j, b):
        pltpu.make_async_copy(new_hbm.at[src_v.at[pl.ds(j * C, C)]],
                              ubufs[b], sems[b]).wait()
        pltpu.make_async_copy(raw_hbm.at[dst_v.at[pl.ds(j * C, C)]],
                              vbufs[b], sems[b]).wait()

    fire(0, 0)
    pltpu.async_copy(new_hbm.at[src_v.at[pl.ds(NCHUNK * C, TAIL)]], ut, semt)
    pltpu.async_copy(raw_hbm.at[dst_v.at[pl.ds(NCHUNK * C, TAIL)]], vt, semt)

    iota = lax.iota(jnp.int32, 16)

    def dot_group(ub, vb, idx_e):
        acc0 = jnp.zeros((16,), jnp.float32)

        def feat_body(w, carry):
            acc_lo, acc_hi = carry
            idx_w = (iota + w) % 16 + (w - w % 16)
            uw = plsc.load_gather(ub, [idx_e, idx_w])
            vw = plsc.load_gather(vb, [idx_e, idx_w])
            p = plsc.bitcast(uw, jnp.bfloat16) * plsc.bitcast(vw, jnp.bfloat16)
            lo, hi = plsc.unpack(p, format=plsc.PackFormat.INTERLEAVED)
            return (acc_lo + lo, acc_hi + hi)

        acc_lo, acc_hi = pl.loop(0, DW, init_carry=(acc0, acc0),
                                 unroll=8)(feat_body)
        return acc_lo + acc_hi

    def compute(j, b):
        for g in range(C // 16):
            idx_e = iota + (g * 16)
            acc = dot_group(ubufs[b], vbufs[b], idx_e)
            scores_v[pl.ds(j * C + g * 16, 16)] = acc

    def outer(j0):
        for b in range(2):
            j = j0 + b

            @pl.when(j < NCHUNK - 1)
            def _():
                fire(j + 1, 1 - b)

            wait(j, b)
            compute(j, b)

    pl.loop(0, NCHUNK, step=2)(outer)

    pltpu.make_async_copy(new_hbm.at[src_v.at[pl.ds(NCHUNK * C, TAIL)]],
                          ut, semt).wait()
    pltpu.make_async_copy(raw_hbm.at[dst_v.at[pl.ds(NCHUNK * C, TAIL)]],
                          vt, semt).wait()
    for g in range(3):
        idx_e = jnp.minimum(iota + (g * 16), TAIL - 1)
        acc = dot_group(ut, vt, idx_e)
        scores_v[pl.ds(NCHUNK * C + g * 16, 16)] = acc

    pltpu.sync_copy(scores_v.at[pl.ds(0, EPW)], out_hbm.at[pl.ds(base, EPW)])


@jax.jit
def _run(new_ft, raw_ft, ei):
    mesh = plsc.VectorSubcoreMesh(core_axis_name="c", subcore_axis_name="s")
    kfn = pl.kernel(
        _sc_body,
        out_type=jax.ShapeDtypeStruct((N_EDGES,), jnp.float32),
        mesh=mesh,
        compiler_params=pltpu.CompilerParams(use_tc_tiling_on_sc=False,
                                             needs_layout_passes=False),
        scratch_types=[
            pltpu.VMEM((EPW,), jnp.int32),
            pltpu.VMEM((EPW,), jnp.int32),
            pltpu.VMEM((C, DW), jnp.int32),
            pltpu.VMEM((C, DW), jnp.int32),
            pltpu.VMEM((C, DW), jnp.int32),
            pltpu.VMEM((C, DW), jnp.int32),
            pltpu.VMEM((TAIL, DW), jnp.int32),
            pltpu.VMEM((TAIL, DW), jnp.int32),
            pltpu.VMEM((EPW + 8,), jnp.float32),
            pltpu.SemaphoreType.DMA,
            pltpu.SemaphoreType.DMA,
            pltpu.SemaphoreType.DMA,
        ],
    )
    return kfn(new_ft, raw_ft, ei)


def _pack_bf16(x):
    u = jax.lax.bitcast_convert_type(x, jnp.uint32)
    b = (u + 0x8000) >> 16
    packed = b[:, :DW] | (b[:, DW:] << 16)
    return jax.lax.bitcast_convert_type(packed, jnp.int32)


def kernel(new_ft, raw_ft, edge_index):
    ei = edge_index.astype(jnp.int32)
    scores = _run(_pack_bf16(new_ft), _pack_bf16(raw_ft), ei)
    return scores.reshape(N_EDGES, 1)
